# baseline (speedup 1.0000x reference)
import functools

import jax
import jax.numpy as jnp
from jax import lax
from jax.experimental import pallas as pl
from jax.experimental.pallas import tpu as pltpu
from jax.experimental.pallas import tpu_sc as plsc

B, P, Q, D = 8, 4096, 4096, 256
QT = 256
NQT = Q // QT
LCH = 128
NCHK = P // LCH


def _argmin_body(p_ref, q_ref, out_ref, pnorm_s):
    b = pl.program_id(0)
    qt = pl.program_id(1)

    p = p_ref[0]

    @pl.when(qt == 0)
    def _():
        pnorm_s[:, :] = jnp.sum(p * p, axis=0, keepdims=True)

    q = q_ref[0]
    q2 = q * -2.0
    qp = jnp.dot(q2, p, preferred_element_type=jnp.float32)
    qnorm = jnp.sum(q * q, axis=1, keepdims=True)
    t1 = qnorm + pnorm_s[:, :]
    scores = t1 + qp

    nodes = []
    for c in range(0, NCHK, 2):
        a = scores[:, c * LCH:(c + 1) * LCH]
        bb = scores[:, (c + 1) * LCH:(c + 2) * LCH]
        t = bb < a
        nodes.append((jnp.where(t, bb, a),
                      jnp.where(t, jnp.int32(c + 1), jnp.int32(c))))
    while len(nodes) > 1:
        nxt = []
        for k in range(0, len(nodes), 2):
            av, ai = nodes[k]
            bv, bi = nodes[k + 1]
            t = bv < av
            nxt.append((jnp.where(t, bv, av), jnp.where(t, bi, ai)))
        nodes = nxt
    m128, c128 = nodes[0]

    lane = lax.broadcasted_iota(jnp.int32, (QT, LCH), 1)
    idx128 = lane * NCHK + c128

    d128 = jnp.sqrt(jnp.maximum(m128, 0.0))
    lmin = jnp.min(d128, axis=1, keepdims=True)
    pick = d128 == lmin
    idx = jnp.min(jnp.where(pick, idx128, P), axis=1)

    out_ref[0, 0] = idx + b * P


def _out_index(b, q):
    return (b * NQT + q, 0, 0)


def _nn_indices(prompt_perm, query_feats, nb):
    return pl.pallas_call(
        _argmin_body,
        grid=(nb, NQT),
        in_specs=[
            pl.BlockSpec((1, D, P), lambda b, q: (b, 0, 0)),
            pl.BlockSpec((1, QT, D), lambda b, q: (b, q, 0)),
        ],
        out_specs=pl.BlockSpec((1, 1, QT), _out_index),
        out_shape=jax.ShapeDtypeStruct((nb * NQT, 1, QT), jnp.int32),
        scratch_shapes=[
            pltpu.VMEM((1, P), jnp.float32),
        ],
        compiler_params=pltpu.CompilerParams(
            dimension_semantics=("arbitrary", "arbitrary"),
        ),
    )(prompt_perm, query_feats)


NC, NS = 2, 16
NW = NC * NS
ROWS = B * Q
ROWS_PER_W = ROWS // NW
CH = 128
NCHUNK = ROWS_PER_W // CH


def _gather_chunks(nchunk):
    def body(table_hbm, idx_hbm, out_hbm, idx_v, rows0, rows1,
             g0, g1, s0, s1):
        wid = lax.axis_index("s") * NC + lax.axis_index("c")
        base = wid * nchunk * CH
        pltpu.sync_copy(idx_hbm.at[pl.ds(wid * nchunk, nchunk)], idx_v)
        bufs = (rows0, rows1)
        gsems = (g0, g1)
        ssems = (s0, s1)
        pltpu.async_copy(table_hbm.at[idx_v.at[0]], bufs[0], gsems[0])
        for i in range(nchunk):
            k = i % 2
            kn = (i + 1) % 2
            if i + 1 < nchunk:
                if i >= 1:
                    pltpu.make_async_copy(
                        bufs[kn], out_hbm.at[pl.ds(base + (i - 1) * CH, CH)],
                        ssems[kn]).wait()
                pltpu.async_copy(
                    table_hbm.at[idx_v.at[i + 1]], bufs[kn], gsems[kn])
            pltpu.make_async_copy(
                table_hbm.at[idx_v.at[i]], bufs[k], gsems[k]).wait()
            pltpu.async_copy(
                bufs[k], out_hbm.at[pl.ds(base + i * CH, CH)], ssems[k])
        if nchunk >= 2:
            k2 = (nchunk - 2) % 2
            pltpu.make_async_copy(
                bufs[k2], out_hbm.at[pl.ds(base + (nchunk - 2) * CH, CH)],
                ssems[k2]).wait()
        k_last = (nchunk - 1) % 2
        pltpu.make_async_copy(
            bufs[k_last],
            out_hbm.at[pl.ds(base + (nchunk - 1) * CH, CH)],
            ssems[k_last]).wait()

    return body


@functools.cache
def _sc_gather(rows):
    nchunk = rows // (NW * CH)
    return pl.kernel(
        _gather_chunks(nchunk),
        out_type=jax.ShapeDtypeStruct((rows, D), jnp.float32),
        mesh=plsc.VectorSubcoreMesh(core_axis_name="c", subcore_axis_name="s"),
        scratch_types=[
            pltpu.VMEM((nchunk, CH), jnp.int32),
            pltpu.VMEM((CH, D), jnp.float32),
            pltpu.VMEM((CH, D), jnp.float32),
            pltpu.SemaphoreType.DMA,
            pltpu.SemaphoreType.DMA,
            pltpu.SemaphoreType.DMA,
            pltpu.SemaphoreType.DMA,
        ],
    )


@jax.jit
def kernel(prompt_feats, query_feats):
    NB = 2
    table = prompt_feats.reshape(ROWS, D)
    outs = []
    for s in range(0, B, NB):
        pf = lax.slice_in_dim(prompt_feats, s, s + NB, axis=0)
        qf = lax.slice_in_dim(query_feats, s, s + NB, axis=0)
        prompt_t = pf.transpose(0, 2, 1)
        prompt_perm = (prompt_t.reshape(NB, D, LCH, NCHK)
                       .transpose(0, 1, 3, 2).reshape(NB, D, P))
        nn_idx = _nn_indices(prompt_perm, qf, NB)
        rows = NB * Q
        idx2d = nn_idx.reshape(rows // CH, CH) + s * P
        outs.append(_sc_gather(rows)(table, idx2d))
    return jnp.concatenate(outs, axis=0).reshape(B, Q, D)


# speedup vs baseline: 1.3923x; 1.3923x over previous
import functools

import jax
import jax.numpy as jnp
from jax import lax
from jax.experimental import pallas as pl
from jax.experimental.pallas import tpu as pltpu
from jax.experimental.pallas import tpu_sc as plsc

B, P, Q, D = 8, 4096, 4096, 256
QT = 512
NQT = Q // QT
LCH = 128
NCHK = P // LCH


def _argmin_body(p_ref, q_ref, out_ref, pnorm_s):
    b = pl.program_id(0)
    qt = pl.program_id(1)

    p = p_ref[0]

    @pl.when(qt == 0)
    def _():
        pnorm_s[:, :] = jnp.sum(p * p, axis=0, keepdims=True)

    q = q_ref[0]
    q2 = q * -2.0
    qp = jnp.dot(q2, p, preferred_element_type=jnp.float32)
    qnorm = jnp.sum(q * q, axis=1, keepdims=True)
    t1 = qnorm + pnorm_s[:, :]
    scores = t1 + qp

    nodes = []
    for c in range(0, NCHK, 2):
        a = scores[:, c * LCH:(c + 1) * LCH]
        bb = scores[:, (c + 1) * LCH:(c + 2) * LCH]
        t = bb < a
        nodes.append((jnp.where(t, bb, a),
                      jnp.where(t, jnp.int32(c + 1), jnp.int32(c))))
    while len(nodes) > 1:
        nxt = []
        for k in range(0, len(nodes), 2):
            av, ai = nodes[k]
            bv, bi = nodes[k + 1]
            t = bv < av
            nxt.append((jnp.where(t, bv, av), jnp.where(t, bi, ai)))
        nodes = nxt
    m128, c128 = nodes[0]

    lane = lax.broadcasted_iota(jnp.int32, (QT, LCH), 1)
    idx128 = lane * NCHK + c128

    d128 = jnp.sqrt(jnp.maximum(m128, 0.0))
    lmin = jnp.min(d128, axis=1, keepdims=True)
    pick = d128 == lmin
    idx = jnp.min(jnp.where(pick, idx128, P), axis=1)

    out_ref[0, 0] = idx + b * P


def _out_index(b, q):
    return (b * NQT + q, 0, 0)


def _nn_indices(prompt_perm, query_feats, nb):
    return pl.pallas_call(
        _argmin_body,
        grid=(nb, NQT),
        in_specs=[
            pl.BlockSpec((1, D, P), lambda b, q: (b, 0, 0)),
            pl.BlockSpec((1, QT, D), lambda b, q: (b, q, 0)),
        ],
        out_specs=pl.BlockSpec((1, 1, QT), _out_index),
        out_shape=jax.ShapeDtypeStruct((nb * NQT, 1, QT), jnp.int32),
        scratch_shapes=[
            pltpu.VMEM((1, P), jnp.float32),
        ],
        compiler_params=pltpu.CompilerParams(
            dimension_semantics=("arbitrary", "arbitrary"),
        ),
    )(prompt_perm, query_feats)


NC, NS = 2, 16
NW = NC * NS
ROWS = B * Q
ROWS_PER_W = ROWS // NW
CH = 128
NCHUNK = ROWS_PER_W // CH


def _gather_chunks(nchunk):
    def body(table_hbm, idx_hbm, out_hbm, idx_v, rows0, rows1,
             g0, g1, s0, s1):
        wid = lax.axis_index("s") * NC + lax.axis_index("c")
        base = wid * nchunk * CH
        pltpu.sync_copy(idx_hbm.at[pl.ds(wid * nchunk, nchunk)], idx_v)
        bufs = (rows0, rows1)
        gsems = (g0, g1)
        ssems = (s0, s1)
        pltpu.async_copy(table_hbm.at[idx_v.at[0]], bufs[0], gsems[0])
        for i in range(nchunk):
            k = i % 2
            kn = (i + 1) % 2
            if i + 1 < nchunk:
                if i >= 1:
                    pltpu.make_async_copy(
                        bufs[kn], out_hbm.at[pl.ds(base + (i - 1) * CH, CH)],
                        ssems[kn]).wait()
                pltpu.async_copy(
                    table_hbm.at[idx_v.at[i + 1]], bufs[kn], gsems[kn])
            pltpu.make_async_copy(
                table_hbm.at[idx_v.at[i]], bufs[k], gsems[k]).wait()
            pltpu.async_copy(
                bufs[k], out_hbm.at[pl.ds(base + i * CH, CH)], ssems[k])
        if nchunk >= 2:
            k2 = (nchunk - 2) % 2
            pltpu.make_async_copy(
                bufs[k2], out_hbm.at[pl.ds(base + (nchunk - 2) * CH, CH)],
                ssems[k2]).wait()
        k_last = (nchunk - 1) % 2
        pltpu.make_async_copy(
            bufs[k_last],
            out_hbm.at[pl.ds(base + (nchunk - 1) * CH, CH)],
            ssems[k_last]).wait()

    return body


@functools.cache
def _sc_gather(rows):
    nchunk = rows // (NW * CH)
    return pl.kernel(
        _gather_chunks(nchunk),
        out_type=jax.ShapeDtypeStruct((rows, D), jnp.float32),
        mesh=plsc.VectorSubcoreMesh(core_axis_name="c", subcore_axis_name="s"),
        scratch_types=[
            pltpu.VMEM((nchunk, CH), jnp.int32),
            pltpu.VMEM((CH, D), jnp.float32),
            pltpu.VMEM((CH, D), jnp.float32),
            pltpu.SemaphoreType.DMA,
            pltpu.SemaphoreType.DMA,
            pltpu.SemaphoreType.DMA,
            pltpu.SemaphoreType.DMA,
        ],
    )


@jax.jit
def kernel(prompt_feats, query_feats):
    prompt_t = prompt_feats.transpose(0, 2, 1)
    prompt_perm = (prompt_t.reshape(B, D, LCH, NCHK)
                   .transpose(0, 1, 3, 2).reshape(B, D, P))
    nn_idx = _nn_indices(prompt_perm, query_feats, B)
    idx2d = nn_idx.reshape(ROWS // CH, CH)
    table = prompt_feats.reshape(ROWS, D)
    out = _sc_gather(ROWS)(table, idx2d)
    return out.reshape(B, Q, D)


# speedup vs baseline: 1.4417x; 1.0355x over previous
import functools

import jax
import jax.numpy as jnp
from jax import lax
from jax.experimental import pallas as pl
from jax.experimental.pallas import tpu as pltpu
from jax.experimental.pallas import tpu_sc as plsc

B, P, Q, D = 8, 4096, 4096, 256
QT = 1024
NQT = Q // QT
LCH = 128
NCHK = P // LCH


def _argmin_body(p_ref, q_ref, out_ref, pnorm_s):
    b = pl.program_id(0)
    qt = pl.program_id(1)

    p = p_ref[0]

    @pl.when(qt == 0)
    def _():
        pnorm_s[:, :] = jnp.sum(p * p, axis=0, keepdims=True)

    q = q_ref[0]
    q2 = q * -2.0
    qp = jnp.dot(q2, p, preferred_element_type=jnp.float32)
    qnorm = jnp.sum(q * q, axis=1, keepdims=True)
    t1 = qnorm + pnorm_s[:, :]
    scores = t1 + qp

    nodes = []
    for c in range(0, NCHK, 2):
        a = scores[:, c * LCH:(c + 1) * LCH]
        bb = scores[:, (c + 1) * LCH:(c + 2) * LCH]
        t = bb < a
        nodes.append((jnp.where(t, bb, a),
                      jnp.where(t, jnp.int32(c + 1), jnp.int32(c))))
    while len(nodes) > 1:
        nxt = []
        for k in range(0, len(nodes), 2):
            av, ai = nodes[k]
            bv, bi = nodes[k + 1]
            t = bv < av
            nxt.append((jnp.where(t, bv, av), jnp.where(t, bi, ai)))
        nodes = nxt
    m128, c128 = nodes[0]

    lane = lax.broadcasted_iota(jnp.int32, (QT, LCH), 1)
    idx128 = lane * NCHK + c128

    d128 = jnp.sqrt(jnp.maximum(m128, 0.0))
    lmin = jnp.min(d128, axis=1, keepdims=True)
    pick = d128 == lmin
    idx = jnp.min(jnp.where(pick, idx128, P), axis=1)

    out_ref[0, 0] = idx + b * P


def _out_index(b, q):
    return (b * NQT + q, 0, 0)


def _nn_indices(prompt_perm, query_feats, nb):
    return pl.pallas_call(
        _argmin_body,
        grid=(nb, NQT),
        in_specs=[
            pl.BlockSpec((1, D, P), lambda b, q: (b, 0, 0)),
            pl.BlockSpec((1, QT, D), lambda b, q: (b, q, 0)),
        ],
        out_specs=pl.BlockSpec((1, 1, QT), _out_index),
        out_shape=jax.ShapeDtypeStruct((nb * NQT, 1, QT), jnp.int32),
        scratch_shapes=[
            pltpu.VMEM((1, P), jnp.float32),
        ],
        compiler_params=pltpu.CompilerParams(
            dimension_semantics=("arbitrary", "arbitrary"),
        ),
    )(prompt_perm, query_feats)


NC, NS = 2, 16
NW = NC * NS
ROWS = B * Q
ROWS_PER_W = ROWS // NW
CH = 128
NCHUNK = ROWS_PER_W // CH


def _gather_chunks(nchunk):
    def body(table_hbm, idx_hbm, out_hbm, idx_v, rows0, rows1,
             g0, g1, s0, s1):
        wid = lax.axis_index("s") * NC + lax.axis_index("c")
        base = wid * nchunk * CH
        pltpu.sync_copy(idx_hbm.at[pl.ds(wid * nchunk, nchunk)], idx_v)
        bufs = (rows0, rows1)
        gsems = (g0, g1)
        ssems = (s0, s1)
        pltpu.async_copy(table_hbm.at[idx_v.at[0]], bufs[0], gsems[0])
        for i in range(nchunk):
            k = i % 2
            kn = (i + 1) % 2
            if i + 1 < nchunk:
                if i >= 1:
                    pltpu.make_async_copy(
                        bufs[kn], out_hbm.at[pl.ds(base + (i - 1) * CH, CH)],
                        ssems[kn]).wait()
                pltpu.async_copy(
                    table_hbm.at[idx_v.at[i + 1]], bufs[kn], gsems[kn])
            pltpu.make_async_copy(
                table_hbm.at[idx_v.at[i]], bufs[k], gsems[k]).wait()
            pltpu.async_copy(
                bufs[k], out_hbm.at[pl.ds(base + i * CH, CH)], ssems[k])
        if nchunk >= 2:
            k2 = (nchunk - 2) % 2
            pltpu.make_async_copy(
                bufs[k2], out_hbm.at[pl.ds(base + (nchunk - 2) * CH, CH)],
                ssems[k2]).wait()
        k_last = (nchunk - 1) % 2
        pltpu.make_async_copy(
            bufs[k_last],
            out_hbm.at[pl.ds(base + (nchunk - 1) * CH, CH)],
            ssems[k_last]).wait()

    return body


@functools.cache
def _sc_gather(rows):
    nchunk = rows // (NW * CH)
    return pl.kernel(
        _gather_chunks(nchunk),
        out_type=jax.ShapeDtypeStruct((rows, D), jnp.float32),
        mesh=plsc.VectorSubcoreMesh(core_axis_name="c", subcore_axis_name="s"),
        scratch_types=[
            pltpu.VMEM((nchunk, CH), jnp.int32),
            pltpu.VMEM((CH, D), jnp.float32),
            pltpu.VMEM((CH, D), jnp.float32),
            pltpu.SemaphoreType.DMA,
            pltpu.SemaphoreType.DMA,
            pltpu.SemaphoreType.DMA,
            pltpu.SemaphoreType.DMA,
        ],
    )


@jax.jit
def kernel(prompt_feats, query_feats):
    prompt_t = prompt_feats.transpose(0, 2, 1)
    prompt_perm = (prompt_t.reshape(B, D, LCH, NCHK)
                   .transpose(0, 1, 3, 2).reshape(B, D, P))
    nn_idx = _nn_indices(prompt_perm, query_feats, B)
    idx2d = nn_idx.reshape(ROWS // CH, CH)
    table = prompt_feats.reshape(ROWS, D)
    out = _sc_gather(ROWS)(table, idx2d)
    return out.reshape(B, Q, D)


# speedup vs baseline: 1.5615x; 1.0831x over previous
import functools

import jax
import jax.numpy as jnp
from jax import lax
from jax.experimental import pallas as pl
from jax.experimental.pallas import tpu as pltpu
from jax.experimental.pallas import tpu_sc as plsc

B, P, Q, D = 8, 4096, 4096, 256
QT = 1024
NQT = Q // QT
LCH = 128
NCHK = P // LCH


def _argmin_body(p_ref, q_ref, out_ref, pnorm_s):
    b = pl.program_id(0)
    qt = pl.program_id(1)

    p = p_ref[0]

    @pl.when(qt == 0)
    def _():
        pnorm_s[:, :] = jnp.sum(p * p, axis=0, keepdims=True)

    q = q_ref[0]
    q2 = q * -2.0
    qp = jnp.dot(q2, p, preferred_element_type=jnp.float32)
    qnorm = jnp.sum(q * q, axis=1, keepdims=True)
    t1 = qnorm + pnorm_s[:, :]
    scores = t1 + qp

    nodes = []
    for c in range(0, NCHK, 2):
        a = scores[:, c * LCH:(c + 1) * LCH]
        bb = scores[:, (c + 1) * LCH:(c + 2) * LCH]
        t = bb < a
        nodes.append((jnp.where(t, bb, a),
                      jnp.where(t, jnp.int32(c + 1), jnp.int32(c))))
    while len(nodes) > 1:
        nxt = []
        for k in range(0, len(nodes), 2):
            av, ai = nodes[k]
            bv, bi = nodes[k + 1]
            t = bv < av
            nxt.append((jnp.where(t, bv, av), jnp.where(t, bi, ai)))
        nodes = nxt
    m128, c128 = nodes[0]

    mT = m128.T
    cT = c128.T

    lane = lax.broadcasted_iota(jnp.int32, (LCH, QT), 0)
    idxT = lane * NCHK + cT

    dT = jnp.sqrt(jnp.maximum(mT, 0.0))
    lmin = jnp.min(dT, axis=0, keepdims=True)
    pick = dT == lmin
    idx = jnp.min(jnp.where(pick, idxT, P), axis=0)

    out_ref[0, 0] = idx + b * P


def _out_index(b, q):
    return (b * NQT + q, 0, 0)


def _nn_indices(prompt_perm, query_feats, nb):
    return pl.pallas_call(
        _argmin_body,
        grid=(nb, NQT),
        in_specs=[
            pl.BlockSpec((1, D, P), lambda b, q: (b, 0, 0)),
            pl.BlockSpec((1, QT, D), lambda b, q: (b, q, 0)),
        ],
        out_specs=pl.BlockSpec((1, 1, QT), _out_index),
        out_shape=jax.ShapeDtypeStruct((nb * NQT, 1, QT), jnp.int32),
        scratch_shapes=[
            pltpu.VMEM((1, P), jnp.float32),
        ],
        compiler_params=pltpu.CompilerParams(
            dimension_semantics=("arbitrary", "arbitrary"),
        ),
    )(prompt_perm, query_feats)


NC, NS = 2, 16
NW = NC * NS
ROWS = B * Q
ROWS_PER_W = ROWS // NW
CH = 128
NCHUNK = ROWS_PER_W // CH


def _gather_chunks(nchunk):
    def body(table_hbm, idx_hbm, out_hbm, idx_v, rows0, rows1,
             g0, g1, s0, s1):
        wid = lax.axis_index("s") * NC + lax.axis_index("c")
        base = wid * nchunk * CH
        pltpu.sync_copy(idx_hbm.at[pl.ds(wid * nchunk, nchunk)], idx_v)
        bufs = (rows0, rows1)
        gsems = (g0, g1)
        ssems = (s0, s1)
        pltpu.async_copy(table_hbm.at[idx_v.at[0]], bufs[0], gsems[0])
        for i in range(nchunk):
            k = i % 2
            kn = (i + 1) % 2
            if i + 1 < nchunk:
                if i >= 1:
                    pltpu.make_async_copy(
                        bufs[kn], out_hbm.at[pl.ds(base + (i - 1) * CH, CH)],
                        ssems[kn]).wait()
                pltpu.async_copy(
                    table_hbm.at[idx_v.at[i + 1]], bufs[kn], gsems[kn])
            pltpu.make_async_copy(
                table_hbm.at[idx_v.at[i]], bufs[k], gsems[k]).wait()
            pltpu.async_copy(
                bufs[k], out_hbm.at[pl.ds(base + i * CH, CH)], ssems[k])
        if nchunk >= 2:
            k2 = (nchunk - 2) % 2
            pltpu.make_async_copy(
                bufs[k2], out_hbm.at[pl.ds(base + (nchunk - 2) * CH, CH)],
                ssems[k2]).wait()
        k_last = (nchunk - 1) % 2
        pltpu.make_async_copy(
            bufs[k_last],
            out_hbm.at[pl.ds(base + (nchunk - 1) * CH, CH)],
            ssems[k_last]).wait()

    return body


@functools.cache
def _sc_gather(rows):
    nchunk = rows // (NW * CH)
    return pl.kernel(
        _gather_chunks(nchunk),
        out_type=jax.ShapeDtypeStruct((rows, D), jnp.float32),
        mesh=plsc.VectorSubcoreMesh(core_axis_name="c", subcore_axis_name="s"),
        scratch_types=[
            pltpu.VMEM((nchunk, CH), jnp.int32),
            pltpu.VMEM((CH, D), jnp.float32),
            pltpu.VMEM((CH, D), jnp.float32),
            pltpu.SemaphoreType.DMA,
            pltpu.SemaphoreType.DMA,
            pltpu.SemaphoreType.DMA,
            pltpu.SemaphoreType.DMA,
        ],
    )


@jax.jit
def kernel(prompt_feats, query_feats):
    prompt_t = prompt_feats.transpose(0, 2, 1)
    prompt_perm = (prompt_t.reshape(B, D, LCH, NCHK)
                   .transpose(0, 1, 3, 2).reshape(B, D, P))
    nn_idx = _nn_indices(prompt_perm, query_feats, B)
    idx2d = nn_idx.reshape(ROWS // CH, CH)
    table = prompt_feats.reshape(ROWS, D)
    out = _sc_gather(ROWS)(table, idx2d)
    return out.reshape(B, Q, D)


# speedup vs baseline: 2.0719x; 1.3269x over previous
import functools

import jax
import jax.numpy as jnp
from jax import lax
from jax.experimental import pallas as pl
from jax.experimental.pallas import tpu as pltpu
from jax.experimental.pallas import tpu_sc as plsc

B, P, Q, D = 8, 4096, 4096, 256
QT = 1024
NQT = Q // QT
LCH = 128
NCHK = P // LCH


NRB = P // 8
NRED = 64


def _argmin_body(p_ref, q_ref, out_ref, pnorm_s):
    b = pl.program_id(0)
    qt = pl.program_id(1)

    po = p_ref[0]

    @pl.when(qt == 0)
    def _():
        b2 = jnp.sum(po * po, axis=1, keepdims=True)
        pnorm_s[:, :] = jnp.broadcast_to(b2, (P, QT))

    q = q_ref[0]
    q2t = (q * -2.0).T
    qp = jnp.dot(po, q2t, preferred_element_type=jnp.float32)

    qnorm = jnp.sum(q * q, axis=1)[None, :]

    def srow(r0, nrows):
        t1 = pnorm_s[pl.ds(r0, nrows), :] + qnorm
        return t1 + qp[r0:r0 + nrows, :]

    nodes = []
    step = NRB // NRED
    for g in range(NRED):
        base_blk = g * step
        av = srow(base_blk * 8, 8)
        ai = jnp.full((8, QT), base_blk, jnp.int32)
        for j in range(1, step):
            blk = base_blk + j
            bv = srow(blk * 8, 8)
            t = bv < av
            av = jnp.where(t, bv, av)
            ai = jnp.where(t, blk, ai)
        nodes.append((av, ai))

    dnodes = [(jnp.sqrt(jnp.maximum(v, 0.0)), i) for v, i in nodes]
    while len(dnodes) > 1:
        nxt = []
        for k in range(0, len(dnodes), 2):
            av, ai = dnodes[k]
            bv, bi = dnodes[k + 1]
            t = bv < av
            nxt.append((jnp.where(t, bv, av), jnp.where(t, bi, ai)))
        dnodes = nxt
    d8, i8 = dnodes[0]

    sub = lax.broadcasted_iota(jnp.int32, (8, QT), 0)
    idx8 = i8 * 8 + sub
    dmin = jnp.min(d8, axis=0, keepdims=True)
    pick = d8 == dmin
    idx = jnp.min(jnp.where(pick, idx8, P), axis=0)

    out_ref[0, 0] = idx + b * P


def _out_index(b, q):
    return (b * NQT + q, 0, 0)


def _nn_indices(prompt_feats, query_feats, nb):
    return pl.pallas_call(
        _argmin_body,
        grid=(nb, NQT),
        in_specs=[
            pl.BlockSpec((1, P, D), lambda b, q: (b, 0, 0)),
            pl.BlockSpec((1, QT, D), lambda b, q: (b, q, 0)),
        ],
        out_specs=pl.BlockSpec((1, 1, QT), _out_index),
        out_shape=jax.ShapeDtypeStruct((nb * NQT, 1, QT), jnp.int32),
        scratch_shapes=[
            pltpu.VMEM((P, QT), jnp.float32),
        ],
        compiler_params=pltpu.CompilerParams(
            dimension_semantics=("arbitrary", "arbitrary"),
        ),
    )(prompt_feats, query_feats)


NC, NS = 2, 16
NW = NC * NS
ROWS = B * Q
ROWS_PER_W = ROWS // NW
CH = 128
NCHUNK = ROWS_PER_W // CH


def _gather_chunks(nchunk):
    def body(table_hbm, idx_hbm, out_hbm, idx_v, rows0, rows1,
             g0, g1, s0, s1):
        wid = lax.axis_index("s") * NC + lax.axis_index("c")
        base = wid * nchunk * CH
        pltpu.sync_copy(idx_hbm.at[pl.ds(wid * nchunk, nchunk)], idx_v)
        bufs = (rows0, rows1)
        gsems = (g0, g1)
        ssems = (s0, s1)
        pltpu.async_copy(table_hbm.at[idx_v.at[0]], bufs[0], gsems[0])
        for i in range(nchunk):
            k = i % 2
            kn = (i + 1) % 2
            if i + 1 < nchunk:
                if i >= 1:
                    pltpu.make_async_copy(
                        bufs[kn], out_hbm.at[pl.ds(base + (i - 1) * CH, CH)],
                        ssems[kn]).wait()
                pltpu.async_copy(
                    table_hbm.at[idx_v.at[i + 1]], bufs[kn], gsems[kn])
            pltpu.make_async_copy(
                table_hbm.at[idx_v.at[i]], bufs[k], gsems[k]).wait()
            pltpu.async_copy(
                bufs[k], out_hbm.at[pl.ds(base + i * CH, CH)], ssems[k])
        if nchunk >= 2:
            k2 = (nchunk - 2) % 2
            pltpu.make_async_copy(
                bufs[k2], out_hbm.at[pl.ds(base + (nchunk - 2) * CH, CH)],
                ssems[k2]).wait()
        k_last = (nchunk - 1) % 2
        pltpu.make_async_copy(
            bufs[k_last],
            out_hbm.at[pl.ds(base + (nchunk - 1) * CH, CH)],
            ssems[k_last]).wait()

    return body


@functools.cache
def _sc_gather(rows):
    nchunk = rows // (NW * CH)
    return pl.kernel(
        _gather_chunks(nchunk),
        out_type=jax.ShapeDtypeStruct((rows, D), jnp.float32),
        mesh=plsc.VectorSubcoreMesh(core_axis_name="c", subcore_axis_name="s"),
        scratch_types=[
            pltpu.VMEM((nchunk, CH), jnp.int32),
            pltpu.VMEM((CH, D), jnp.float32),
            pltpu.VMEM((CH, D), jnp.float32),
            pltpu.SemaphoreType.DMA,
            pltpu.SemaphoreType.DMA,
            pltpu.SemaphoreType.DMA,
            pltpu.SemaphoreType.DMA,
        ],
    )


@jax.jit
def kernel(prompt_feats, query_feats):
    nn_idx = _nn_indices(prompt_feats, query_feats, B)
    idx2d = nn_idx.reshape(ROWS // CH, CH)
    table = prompt_feats.reshape(ROWS, D)
    out = _sc_gather(ROWS)(table, idx2d)
    return out.reshape(B, Q, D)


# speedup vs baseline: 2.2647x; 1.0931x over previous
import functools

import jax
import jax.numpy as jnp
from jax import lax
from jax.experimental import pallas as pl
from jax.experimental.pallas import tpu as pltpu
from jax.experimental.pallas import tpu_sc as plsc

B, P, Q, D = 8, 4096, 4096, 256
QT = 1024
NQT = Q // QT
LCH = 128
NCHK = P // LCH


NRB = P // 8
NRED = 32


def _argmin_body(p_ref, q_ref, out_ref, pnorm_s):
    b = pl.program_id(0)
    qt = pl.program_id(1)

    po = p_ref[0]

    @pl.when(qt == 0)
    def _():
        b2 = jnp.sum(po * po, axis=1, keepdims=True)
        pnorm_s[:, :] = jnp.broadcast_to(b2, (P, QT))

    q = q_ref[0]
    q2t = (q * -2.0).T
    qp = jnp.dot(po, q2t, preferred_element_type=jnp.float32)

    qnorm = jnp.sum(q * q, axis=1)[None, :]

    def srow(r0, nrows):
        t1 = pnorm_s[pl.ds(r0, nrows), :] + qnorm
        return t1 + qp[r0:r0 + nrows, :]

    nodes = []
    step = NRB // NRED
    for g in range(NRED):
        base_blk = g * step
        av = srow(base_blk * 8, 8)
        ai = jnp.full((8, QT), base_blk, jnp.int32)
        for j in range(1, step):
            blk = base_blk + j
            bv = srow(blk * 8, 8)
            t = bv < av
            av = jnp.where(t, bv, av)
            ai = jnp.where(t, blk, ai)
        nodes.append((av, ai))

    dnodes = [(jnp.sqrt(jnp.maximum(v, 0.0)), i) for v, i in nodes]
    while len(dnodes) > 1:
        nxt = []
        for k in range(0, len(dnodes), 2):
            av, ai = dnodes[k]
            bv, bi = dnodes[k + 1]
            t = bv < av
            nxt.append((jnp.where(t, bv, av), jnp.where(t, bi, ai)))
        dnodes = nxt
    d8, i8 = dnodes[0]

    sub = lax.broadcasted_iota(jnp.int32, (8, QT), 0)
    idx8 = i8 * 8 + sub
    dmin = jnp.min(d8, axis=0, keepdims=True)
    pick = d8 == dmin
    idx = jnp.min(jnp.where(pick, idx8, P), axis=0)

    out_ref[0, 0] = idx + b * P


def _out_index(b, q):
    return (b * NQT + q, 0, 0)


def _nn_indices(prompt_feats, query_feats, nb):
    return pl.pallas_call(
        _argmin_body,
        grid=(nb, NQT),
        in_specs=[
            pl.BlockSpec((1, P, D), lambda b, q: (b, 0, 0)),
            pl.BlockSpec((1, QT, D), lambda b, q: (b, q, 0)),
        ],
        out_specs=pl.BlockSpec((1, 1, QT), _out_index),
        out_shape=jax.ShapeDtypeStruct((nb * NQT, 1, QT), jnp.int32),
        scratch_shapes=[
            pltpu.VMEM((P, QT), jnp.float32),
        ],
        compiler_params=pltpu.CompilerParams(
            dimension_semantics=("arbitrary", "arbitrary"),
        ),
    )(prompt_feats, query_feats)


NC, NS = 2, 16
NW = NC * NS
ROWS = B * Q
ROWS_PER_W = ROWS // NW
CH = 128
NCHUNK = ROWS_PER_W // CH


def _gather_chunks(nchunk):
    def body(table_hbm, idx_hbm, out_hbm, idx_v, rows0, rows1,
             g0, g1, s0, s1):
        wid = lax.axis_index("s") * NC + lax.axis_index("c")
        base = wid * nchunk * CH
        pltpu.sync_copy(idx_hbm.at[pl.ds(wid * nchunk, nchunk)], idx_v)
        bufs = (rows0, rows1)
        gsems = (g0, g1)
        ssems = (s0, s1)
        pltpu.async_copy(table_hbm.at[idx_v.at[0]], bufs[0], gsems[0])
        for i in range(nchunk):
            k = i % 2
            kn = (i + 1) % 2
            if i + 1 < nchunk:
                if i >= 1:
                    pltpu.make_async_copy(
                        bufs[kn], out_hbm.at[pl.ds(base + (i - 1) * CH, CH)],
                        ssems[kn]).wait()
                pltpu.async_copy(
                    table_hbm.at[idx_v.at[i + 1]], bufs[kn], gsems[kn])
            pltpu.make_async_copy(
                table_hbm.at[idx_v.at[i]], bufs[k], gsems[k]).wait()
            pltpu.async_copy(
                bufs[k], out_hbm.at[pl.ds(base + i * CH, CH)], ssems[k])
        if nchunk >= 2:
            k2 = (nchunk - 2) % 2
            pltpu.make_async_copy(
                bufs[k2], out_hbm.at[pl.ds(base + (nchunk - 2) * CH, CH)],
                ssems[k2]).wait()
        k_last = (nchunk - 1) % 2
        pltpu.make_async_copy(
            bufs[k_last],
            out_hbm.at[pl.ds(base + (nchunk - 1) * CH, CH)],
            ssems[k_last]).wait()

    return body


@functools.cache
def _sc_gather(rows):
    nchunk = rows // (NW * CH)
    return pl.kernel(
        _gather_chunks(nchunk),
        out_type=jax.ShapeDtypeStruct((rows, D), jnp.float32),
        mesh=plsc.VectorSubcoreMesh(core_axis_name="c", subcore_axis_name="s"),
        scratch_types=[
            pltpu.VMEM((nchunk, CH), jnp.int32),
            pltpu.VMEM((CH, D), jnp.float32),
            pltpu.VMEM((CH, D), jnp.float32),
            pltpu.SemaphoreType.DMA,
            pltpu.SemaphoreType.DMA,
            pltpu.SemaphoreType.DMA,
            pltpu.SemaphoreType.DMA,
        ],
    )


@jax.jit
def kernel(prompt_feats, query_feats):
    nn_idx = _nn_indices(prompt_feats, query_feats, B)
    idx2d = nn_idx.reshape(ROWS // CH, CH)
    table = prompt_feats.reshape(ROWS, D)
    out = _sc_gather(ROWS)(table, idx2d)
    return out.reshape(B, Q, D)
